# baseline (speedup 1.0000x reference)
import functools

import jax
import jax.numpy as jnp
from jax import lax
from jax.experimental import pallas as pl
from jax.experimental.pallas import tpu as pltpu
from jax.experimental.pallas import tpu_sc as plsc

_NC = 2
_NS = 16
_NW = _NC * _NS


@functools.lru_cache(maxsize=None)
def _make_tgather(hist: int, batch: int, d: int):
    bpw = batch // _NW
    du = d // 16
    n_pairs = hist // 2
    assert hist % 2 == 0 and batch % _NW == 0 and d % 16 == 0 and bpw % 8 == 0

    mesh = plsc.VectorSubcoreMesh(core_axis_name="c", subcore_axis_name="s")

    @functools.partial(
        pl.kernel,
        mesh=mesh,
        compiler_params=pltpu.CompilerParams(
            use_tc_tiling_on_sc=False, needs_layout_passes=False
        ),
        out_type=jax.ShapeDtypeStruct((hist, d, batch), jnp.float32),
        scratch_types=[
            pltpu.VMEM((hist, bpw), jnp.int32),
            pltpu.VMEM((bpw, d), jnp.float32),
            pltpu.VMEM((bpw, d), jnp.float32),
            pltpu.VMEM((d, bpw), jnp.float32),
            pltpu.VMEM((d, bpw), jnp.float32),
            pltpu.SemaphoreType.DMA,
            pltpu.SemaphoreType.DMA,
            pltpu.SemaphoreType.DMA,
            pltpu.SemaphoreType.DMA,
        ],
    )
    def tgather_kernel(idx_hbm, table_hbm, out_hbm, idxb, g0, g1, t0, t1,
                       gsem0, gsem1, osem0, osem1):
        wid = lax.axis_index("s") * _NC + lax.axis_index("c")
        b0 = wid * bpw
        iota16 = lax.iota(jnp.int32, 16)
        row_ids = [iota16 + 16 * u for u in range(du)]

        def fire_gather(h, g, sem):
            pltpu.async_copy(table_hbm.at[idxb.at[h]], g, sem)

        def wait_gather(h, g, sem):
            pltpu.make_async_copy(table_hbm.at[idxb.at[h]], g, sem).wait()

        def fire_out(h, t, sem):
            pltpu.async_copy(t, out_hbm.at[h, :, pl.ds(b0, bpw)], sem)

        def wait_out(h, t, sem):
            pltpu.make_async_copy(
                t, out_hbm.at[h, :, pl.ds(b0, bpw)], sem
            ).wait()

        def transpose(g, t):
            def tb(i, carry):
                b = i * 8
                for j in range(8):
                    col = jnp.zeros((16,), jnp.int32) + (b + j)
                    for u in range(du):
                        x = g[b + j, pl.ds(16 * u, 16)]
                        plsc.store_scatter(t, [row_ids[u], col], x)
                return carry

            lax.fori_loop(0, bpw // 8, tb, 0)

        pltpu.sync_copy(idx_hbm.at[:, pl.ds(b0, bpw)], idxb)
        fire_gather(0, g0, gsem0)

        def body(t, carry):
            h0 = 2 * t
            h1 = h0 + 1

            @pl.when(t >= 1)
            def _wait_prev_out1():
                wait_out(h1 - 2, t1, osem1)

            fire_gather(h1, g1, gsem1)
            wait_gather(h0, g0, gsem0)

            @pl.when(t >= 1)
            def _wait_prev_out0():
                wait_out(h0 - 2, t0, osem0)

            transpose(g0, t0)
            fire_out(h0, t0, osem0)

            @pl.when(t < n_pairs - 1)
            def _refill_g0():
                fire_gather(h0 + 2, g0, gsem0)

            wait_gather(h1, g1, gsem1)
            transpose(g1, t1)
            fire_out(h1, t1, osem1)
            return carry

        lax.fori_loop(0, n_pairs, body, 0)
        wait_out(hist - 2, t0, osem0)
        wait_out(hist - 1, t1, osem1)

    return tgather_kernel


def kernel(indices, table):
    batch, hist = indices.shape
    d = table.shape[1]
    idx_t = jnp.transpose(indices).astype(jnp.int32)
    out_t = _make_tgather(hist, batch, d)(idx_t, table)
    return jnp.transpose(out_t, (2, 0, 1))


# speedup vs baseline: 1.8572x; 1.8572x over previous
import functools

import jax
import jax.numpy as jnp
from jax import lax
from jax.experimental import pallas as pl
from jax.experimental.pallas import tpu as pltpu
from jax.experimental.pallas import tpu_sc as plsc

_NC = 2
_NS = 16
_NW = _NC * _NS


@functools.lru_cache(maxsize=None)
def _make_gather(batch: int, hist: int, d: int, bchunk: int):
    b_per_w = batch // _NW
    rows_per_w = b_per_w * hist
    n_groups = b_per_w // bchunk
    chunk = bchunk * hist
    n_pairs = n_groups // 2
    assert batch % _NW == 0 and n_groups % 2 == 0

    mesh = plsc.VectorSubcoreMesh(core_axis_name="c", subcore_axis_name="s")

    @functools.partial(
        pl.kernel,
        mesh=mesh,
        compiler_params=pltpu.CompilerParams(use_tc_tiling_on_sc=False),
        out_type=jax.ShapeDtypeStruct((batch, hist, d), jnp.float32),
        scratch_types=[
            pltpu.VMEM((rows_per_w,), jnp.int32),
            pltpu.VMEM((chunk, d), jnp.float32),
            pltpu.VMEM((chunk, d), jnp.float32),
            pltpu.SemaphoreType.DMA,
            pltpu.SemaphoreType.DMA,
            pltpu.SemaphoreType.DMA,
            pltpu.SemaphoreType.DMA,
        ],
    )
    def gather_kernel(idx_hbm, table_hbm, out_hbm, idx_v, rows0, rows1,
                      gsem0, gsem1, osem0, osem1):
        wid = lax.axis_index("s") * _NC + lax.axis_index("c")
        base = wid * rows_per_w
        b_base = wid * b_per_w

        def fire_gather(g, rows_v, gsem):
            pltpu.async_copy(
                table_hbm.at[idx_v.at[pl.ds(g * chunk, chunk)]],
                rows_v,
                gsem,
            )

        def wait_gather(g, rows_v, gsem):
            pltpu.make_async_copy(
                table_hbm.at[idx_v.at[pl.ds(g * chunk, chunk)]],
                rows_v,
                gsem,
            ).wait()

        def fire_out(g, rows_v, osem):
            for j in range(bchunk):
                pltpu.async_copy(
                    rows_v.at[pl.ds(j * hist, hist)],
                    out_hbm.at[b_base + g * bchunk + j],
                    osem,
                )

        def wait_out(g, rows_v, osem):
            for j in range(bchunk):
                pltpu.make_async_copy(
                    rows_v.at[pl.ds(j * hist, hist)],
                    out_hbm.at[b_base + g * bchunk + j],
                    osem,
                ).wait()

        pltpu.sync_copy(idx_hbm.at[pl.ds(base, rows_per_w)], idx_v)
        fire_gather(0, rows0, gsem0)

        def body(t, carry):
            g0 = 2 * t
            g1 = g0 + 1

            @pl.when(t >= 1)
            def _wait_prev_out1():
                wait_out(g1 - 2, rows1, osem1)

            fire_gather(g1, rows1, gsem1)
            wait_gather(g0, rows0, gsem0)
            fire_out(g0, rows0, osem0)

            @pl.when(t < n_pairs - 1)
            def _refill_buf0():
                wait_out(g0, rows0, osem0)
                fire_gather(g0 + 2, rows0, gsem0)

            wait_gather(g1, rows1, gsem1)
            fire_out(g1, rows1, osem1)
            return carry

        lax.fori_loop(0, n_pairs, body, 0)
        wait_out(n_groups - 2, rows0, osem0)
        wait_out(n_groups - 1, rows1, osem1)

    return gather_kernel


def kernel(indices, table):
    batch, hist = indices.shape
    d = table.shape[1]
    flat = indices.reshape(-1).astype(jnp.int32)
    return _make_gather(batch, hist, d, 4)(flat, table)
